# Initial kernel scaffold; baseline (speedup 1.0000x reference)
import jax
import jax.numpy as jnp
from jax.experimental import pallas as pl


def kernel(x1, x2, table):
    raise NotImplementedError("write your pallas kernel here")


# speedup vs baseline: 2.2713x; 2.2713x over previous
import functools

import jax
import jax.numpy as jnp
from jax import lax
from jax.experimental import pallas as pl
from jax.experimental.pallas import tpu as pltpu
from jax.experimental.pallas import tpu_sc as plsc

_LANES = 16


def _sc_mul_kernel(n_positions: int, d: int, chunk: int):
    info = plsc.get_sparse_core_info()
    nw = info.num_cores * info.num_subcores
    per_worker = n_positions // nw
    n_rounds = per_worker // chunk
    assert per_worker % chunk == 0
    assert d % _LANES == 0

    mesh = plsc.VectorSubcoreMesh(core_axis_name="c", subcore_axis_name="s")

    @functools.partial(
        pl.kernel,
        mesh=mesh,
        out_type=jax.ShapeDtypeStruct((n_positions, d), jnp.float32),
        scratch_types=[
            pltpu.VMEM((64, d), jnp.float32),
            pltpu.VMEM((chunk,), jnp.int32),
            pltpu.VMEM((chunk, d), jnp.float32),
            pltpu.VMEM((chunk, d), jnp.float32),
        ],
    )
    def k(x1_hbm, x2_hbm, table_hbm, out_hbm, tab_v, idx_v, x2_v, out_v):
        wid = lax.axis_index("s") * info.num_cores + lax.axis_index("c")
        pltpu.sync_copy(table_hbm, tab_v)

        def round_body(r, carry):
            base = wid * per_worker + r * chunk
            pltpu.sync_copy(x1_hbm.at[pl.ds(base, chunk)], idx_v)
            pltpu.sync_copy(x2_hbm.at[pl.ds(base, chunk)], x2_v)

            def grp_body(g, c2):
                idxvec = idx_v[pl.ds(g * _LANES, _LANES)]
                for k in range(_LANES):
                    i = idxvec[k]
                    p = g * _LANES + k
                    for j in range(d // _LANES):
                        sl = pl.ds(j * _LANES, _LANES)
                        out_v[p, sl] = tab_v[i, sl] * x2_v[p, sl]
                return c2

            lax.fori_loop(0, chunk // _LANES, grp_body, 0)
            pltpu.sync_copy(out_v, out_hbm.at[pl.ds(base, chunk)])
            return carry

        lax.fori_loop(0, n_rounds, round_body, 0)

    return k


def kernel(x1, x2, table):
    b, l, d = x2.shape
    n = b * l
    x1f = x1.reshape(n).astype(jnp.int32)
    x2f = x2.reshape(n, d)
    out = _sc_mul_kernel(n, d, chunk=320)(x1f, x2f, table)
    return out.reshape(b, l, d)


# speedup vs baseline: 3.9548x; 1.7412x over previous
import functools

import jax
import jax.numpy as jnp
from jax import lax
from jax.experimental import pallas as pl
from jax.experimental.pallas import tpu as pltpu
from jax.experimental.pallas import tpu_sc as plsc

_LANES = 16


def _sc_mul_kernel(n_positions: int, d: int, chunk: int):
    info = plsc.get_sparse_core_info()
    nw = info.num_cores * info.num_subcores
    per_worker = n_positions // nw
    n_rounds = per_worker // chunk
    assert per_worker % chunk == 0
    assert d % _LANES == 0
    assert chunk <= 128

    mesh = plsc.VectorSubcoreMesh(core_axis_name="c", subcore_axis_name="s")

    @functools.partial(
        pl.kernel,
        mesh=mesh,
        out_type=jax.ShapeDtypeStruct((n_positions, d), jnp.float32),
        scratch_types=[
            pltpu.VMEM_SHARED((64, d), jnp.float32),
            pltpu.VMEM((chunk,), jnp.int32),
            pltpu.VMEM((chunk, d), jnp.float32),
            pltpu.VMEM((chunk, d), jnp.float32),
            pltpu.VMEM((chunk, d), jnp.float32),
            pltpu.SemaphoreType.DMA,
        ],
    )
    def k(x1_hbm, x2_hbm, table_hbm, out_hbm, tab_v, idx_v, x2_v, rows_v,
          out_v, sem):
        wid = lax.axis_index("s") * info.num_cores + lax.axis_index("c")
        @pl.when(lax.axis_index("s") == 0)
        def _copy_table():
            pltpu.sync_copy(table_hbm, tab_v)

        plsc.subcore_barrier()

        def round_body(r, carry):
            base = wid * per_worker + r * chunk
            pltpu.sync_copy(x1_hbm.at[pl.ds(base, chunk)], idx_v)
            gcp = pltpu.async_copy(tab_v.at[idx_v], rows_v, sem)
            pltpu.sync_copy(x2_hbm.at[pl.ds(base, chunk)], x2_v)
            gcp.wait()

            def pos_body(p, c2):
                for j in range(d // _LANES):
                    sl = pl.ds(j * _LANES, _LANES)
                    out_v[p, sl] = rows_v[p, sl] * x2_v[p, sl]
                return c2

            lax.fori_loop(0, chunk, pos_body, 0)
            pltpu.sync_copy(out_v, out_hbm.at[pl.ds(base, chunk)])
            return carry

        lax.fori_loop(0, n_rounds, round_body, 0)

    return k


def kernel(x1, x2, table):
    b, l, d = x2.shape
    n = b * l
    x1f = x1.reshape(n).astype(jnp.int32)
    x2f = x2.reshape(n, d)
    out = _sc_mul_kernel(n, d, chunk=128)(x1f, x2f, table)
    return out.reshape(b, l, d)
